# baseline (speedup 1.0000x reference)
import functools

import jax
import jax.numpy as jnp
from jax import lax
from jax.experimental import pallas as pl
from jax.experimental.pallas import tpu as pltpu
from jax.experimental.pallas import tpu_sc as plsc

NX = 512
NY = 512
NCELLS = NX * NY
F = 6
NC = 2
NS = 16
NT = NC * NS
CHUNK = 6272
NP_PAD = NT * CHUNK
NWIN = 8
WE = CHUNK * F // NWIN
ACC = F * NCELLS
ACC_SLICE = ACC // NS
ZB = 4096

_mesh = plsc.VectorSubcoreMesh(core_axis_name="c", subcore_axis_name="s")


@functools.partial(
    pl.kernel,
    mesh=_mesh,
    out_type=jax.ShapeDtypeStruct((NC * ACC,), jnp.float32),
    scratch_types=[
        pltpu.VMEM_SHARED((ACC,), jnp.float32),
        pltpu.VMEM((ZB,), jnp.float32),
        pltpu.VMEM((WE,), jnp.int32),
        pltpu.VMEM((WE,), jnp.int32),
        pltpu.VMEM((WE,), jnp.float32),
        pltpu.VMEM((WE,), jnp.float32),
        pltpu.SemaphoreType.DMA,
        pltpu.SemaphoreType.DMA,
        pltpu.SemaphoreType.DMA,
        pltpu.SemaphoreType.DMA,
    ],
)
def _sc_scatter(val_hbm, eidx_hbm, part_hbm,
                acc, zb, iwa, iwb, vwa, vwb, sia, sib, sva, svb):
    c = lax.axis_index("c")
    s = lax.axis_index("s")
    tile = c * NS + s
    base = tile * CHUNK * F
    a0 = s * ACC_SLICE
    iws = (iwa, iwb)
    vws = (vwa, vwb)
    isems = (sia, sib)
    vsems = (sva, svb)

    iloads = [pltpu.async_copy(eidx_hbm.at[pl.ds(base, WE)], iwa, sia)]
    vloads = [pltpu.async_copy(val_hbm.at[pl.ds(base, WE)], vwa, sva)]

    @pl.loop(0, ZB // 16)
    def _(i):
        zb[pl.ds(i * 16, 16)] = jnp.zeros((16,), jnp.float32)

    @pl.loop(0, ACC_SLICE // ZB)
    def _(i):
        pltpu.sync_copy(zb, acc.at[pl.ds(a0 + i * ZB, ZB)])

    plsc.subcore_barrier()

    for w in range(NWIN):
        b = w % 2
        if w + 1 < NWIN:
            off = base + (w + 1) * WE
            iloads.append(pltpu.async_copy(
                eidx_hbm.at[pl.ds(off, WE)], iws[1 - b], isems[1 - b]))
            vloads.append(pltpu.async_copy(
                val_hbm.at[pl.ds(off, WE)], vws[1 - b], vsems[1 - b]))
        iloads[w].wait()
        vloads[w].wait()
        pltpu.sync_copy(vws[b], acc.at[iws[b]], add=True)

    plsc.subcore_barrier()
    pltpu.sync_copy(acc.at[pl.ds(a0, ACC_SLICE)],
                    part_hbm.at[pl.ds(c * ACC + a0, ACC_SLICE)])


_TCROWS = ACC // 128 // F


def _tc_assemble_body(a_ref, b_ref, o_ref):
    s = a_ref[...] + b_ref[...]
    o_ref[...] = s.reshape(1, NX, NY)


_tc_assemble = pl.pallas_call(
    _tc_assemble_body,
    grid=(F,),
    in_specs=[
        pl.BlockSpec((_TCROWS, 128), lambda i: (i, 0)),
        pl.BlockSpec((_TCROWS, 128), lambda i: (i + F, 0)),
    ],
    out_specs=pl.BlockSpec((1, NX, NY), lambda i: (i, 0, 0)),
    out_shape=jax.ShapeDtypeStruct((F, NX, NY), jnp.float32),
)


def kernel(x, indices):
    n = x.shape[0]
    idx = indices.astype(jnp.int32)
    npad = NP_PAD - n
    idx_pad = jnp.concatenate(
        [idx, (jnp.arange(npad, dtype=jnp.int32) * 97) % NCELLS])
    eidx = (idx_pad[:, None]
            + (jnp.arange(F, dtype=jnp.int32) * NCELLS)[None, :]).reshape(-1)
    vals = jnp.concatenate(
        [x.astype(jnp.float32).reshape(-1),
         jnp.zeros((npad * F,), jnp.float32)])
    part = _sc_scatter(vals, eidx)
    part2d = part.reshape(NC * F * _TCROWS, 128)
    return _tc_assemble(part2d, part2d)


# speedup vs baseline: 4.3499x; 4.3499x over previous
import functools

import jax
import jax.numpy as jnp
from jax import lax
from jax.experimental import pallas as pl
from jax.experimental.pallas import tpu as pltpu
from jax.experimental.pallas import tpu_sc as plsc

NX = 512
NY = 512
NCELLS = NX * NY
F = 6
NC = 2
NS = 16
NT = NC * NS
CHUNK = 6272
NP_PAD = NT * CHUNK
ACC = F * NCELLS
ACC_SLICE = ACC // NS
ZB = 4096

_mesh = plsc.VectorSubcoreMesh(core_axis_name="c", subcore_axis_name="s")


@functools.partial(
    pl.kernel,
    mesh=_mesh,
    out_type=jax.ShapeDtypeStruct((NC * ACC,), jnp.float32),
    scratch_types=[
        pltpu.VMEM_SHARED((ACC,), jnp.float32),
        pltpu.VMEM((ZB,), jnp.float32),
        pltpu.VMEM((CHUNK,), jnp.int32),
        pltpu.VMEM((CHUNK,), jnp.int32),
        pltpu.VMEM((CHUNK,), jnp.float32),
        pltpu.VMEM((CHUNK,), jnp.float32),
        pltpu.SemaphoreType.DMA,
        pltpu.SemaphoreType.DMA,
        pltpu.SemaphoreType.DMA,
    ],
)
def _sc_scatter(v0, v1, v2, v3, v4, v5, idx_hbm, part_hbm,
                acc, zb, isha, ishb, vwa, vwb, semz, sla, slb):
    c = lax.axis_index("c")
    s = lax.axis_index("s")
    tile = c * NS + s
    base = tile * CHUNK
    a0 = s * ACC_SLICE
    vfs = (v0, v1, v2, v3, v4, v5)
    bufs = (vwa, vwb)
    ishs = (isha, ishb)
    lsems = (sla, slb)

    idx_load = pltpu.async_copy(idx_hbm.at[pl.ds(base, CHUNK)], isha, slb)
    loads = [pltpu.async_copy(v0.at[pl.ds(base, CHUNK)], vwa, sla)]

    @pl.loop(0, ZB // 16)
    def _(i):
        zb[pl.ds(i * 16, 16)] = jnp.zeros((16,), jnp.float32)

    zcopies = [
        pltpu.async_copy(zb, acc.at[pl.ds(a0 + i * ZB, ZB)], semz)
        for i in range(ACC_SLICE // ZB)
    ]
    for zc in zcopies:
        zc.wait()
    idx_load.wait()
    plsc.subcore_barrier()

    for f in range(F):
        b = f % 2
        if f + 1 < F:
            loads.append(
                pltpu.async_copy(vfs[f + 1].at[pl.ds(base, CHUNK)],
                                 bufs[1 - b], lsems[1 - b]))

            @pl.loop(0, CHUNK // 128)
            def _(i, b=b):
                for j in range(8):
                    sl = pl.ds(i * 128 + j * 16, 16)
                    ishs[1 - b][sl] = ishs[b][sl] + NCELLS

        loads[f].wait()
        pltpu.sync_copy(bufs[b], acc.at[ishs[b]], add=True)

    plsc.subcore_barrier()
    pltpu.sync_copy(acc.at[pl.ds(a0, ACC_SLICE)],
                    part_hbm.at[pl.ds(c * ACC + a0, ACC_SLICE)])


_TCROWS = NCELLS // 128


def _tc_assemble_body(a_ref, b_ref, o_ref):
    s = a_ref[...] + b_ref[...]
    o_ref[...] = s.reshape(1, NX, NY)


_tc_assemble = pl.pallas_call(
    _tc_assemble_body,
    grid=(F,),
    in_specs=[
        pl.BlockSpec((_TCROWS, 128), lambda i: (i, 0)),
        pl.BlockSpec((_TCROWS, 128), lambda i: (i + F, 0)),
    ],
    out_specs=pl.BlockSpec((1, NX, NY), lambda i: (i, 0, 0)),
    out_shape=jax.ShapeDtypeStruct((F, NX, NY), jnp.float32),
)


def kernel(x, indices):
    n = x.shape[0]
    idx = indices.astype(jnp.int32)
    npad = NP_PAD - n
    idx_pad = jnp.concatenate(
        [idx, (jnp.arange(npad, dtype=jnp.int32) * 97) % NCELLS])
    xf = x.astype(jnp.float32)
    zpad = jnp.zeros((npad,), jnp.float32)
    vfs = [jnp.concatenate([xf[:, f], zpad]) for f in range(F)]
    part = _sc_scatter(*vfs, idx_pad)
    part2d = part.reshape(NC * F * _TCROWS, 128)
    return _tc_assemble(part2d, part2d)
